# baseline (speedup 1.0000x reference)
import functools

import jax
import jax.numpy as jnp
from jax import lax
from jax.experimental import pallas as pl
from jax.experimental.pallas import tpu as pltpu
from jax.experimental.pallas import tpu_sc as plsc

Q = 1024
D = 128
KTOT = 100000
W = 2048
S = W // 128
T = (KTOT + W - 1) // W
KPAD = T * W
NB = KPAD // 128
NEG = float("-inf")
BIGI = 2**30

NWORK = 32
RPW = (Q * 8) // NWORK


def _stage_a(q_ref, k_ref, sim_ref, blk_ref, flat_ref, tb_s, rv_s, ri_s):
    t = pl.program_id(0)
    q = q_ref[...]
    qn = q / jnp.maximum(jnp.sqrt(jnp.sum(q * q, axis=1, keepdims=True)), 1e-8)
    k = k_ref[...]
    kn = k / jnp.maximum(jnp.sqrt(jnp.sum(k * k, axis=1, keepdims=True)), 1e-8)
    sim = lax.dot_general(
        qn, kn, (((1,), (1,)), ((), ())),
        preferred_element_type=jnp.float32,
    )
    simt = lax.dot_general(
        kn, qn, (((1,), (1,)), ((), ())),
        preferred_element_type=jnp.float32,
    )

    @pl.when(t < T - 1)
    def _store_full():
        for j in range(S):
            sim_ref[0, :, j, :] = sim[:, j * 128:(j + 1) * 128]
        for j in range(S):
            tb_s[j, :] = jnp.max(simt[j * 128:(j + 1) * 128, :], axis=0)

    @pl.when(t == T - 1)
    def _store_masked():
        col = lax.broadcasted_iota(jnp.int32, (Q, W), 1)
        simm = jnp.where(col + t * W < KTOT, sim, NEG)
        for j in range(S):
            sim_ref[0, :, j, :] = simm[:, j * 128:(j + 1) * 128]
        rowi = lax.broadcasted_iota(jnp.int32, (W, Q), 0)
        simtm = jnp.where(rowi + t * W < KTOT, simt, NEG)
        for j in range(S):
            tb_s[j, :] = jnp.max(simtm[j * 128:(j + 1) * 128, :], axis=0)

    bm = tb_s[...]
    bid = lax.broadcasted_iota(jnp.int32, (S, Q), 0) + t * S
    tv, ti = [], []
    for _ in range(8):
        m = jnp.max(bm, axis=0)
        pick = jnp.min(jnp.where(bm == m[None, :], bid, BIGI), axis=0)
        tv.append(m)
        ti.append(pick)
        bm = jnp.where(bid == pick[None, :], NEG, bm)
    tilev = jnp.stack(tv, axis=0)
    tilei = jnp.stack(ti, axis=0)

    @pl.when(t == 0)
    def _init():
        rv_s[...] = tilev
        ri_s[...] = tilei

    @pl.when(t != 0)
    def _merge():
        cv = jnp.concatenate([rv_s[...], tilev], axis=0)
        ci = jnp.concatenate([ri_s[...], tilei], axis=0)
        mv, mi = [], []
        for _ in range(8):
            m = jnp.max(cv, axis=0)
            pick = jnp.min(jnp.where(cv == m[None, :], ci, BIGI), axis=0)
            mv.append(m)
            mi.append(pick)
            cv = jnp.where(ci == pick[None, :], NEG, cv)
        rv_s[...] = jnp.stack(mv, axis=0)
        ri_s[...] = jnp.stack(mi, axis=0)

    @pl.when(t == T - 1)
    def _emit():
        blk = jnp.transpose(ri_s[...], (1, 0))
        blk_ref[...] = blk
        rows = lax.broadcasted_iota(jnp.int32, (Q, 8), 0)
        flat_ref[...] = (blk // S) * (Q * S) + rows * S + (blk % S)


def _stage_a_call(queries, kp):
    return pl.pallas_call(
        _stage_a,
        grid=(T,),
        in_specs=[
            pl.BlockSpec((Q, D), lambda t: (0, 0)),
            pl.BlockSpec((W, D), lambda t: (t, 0)),
        ],
        out_specs=[
            pl.BlockSpec((1, Q, S, 128), lambda t: (t, 0, 0, 0)),
            pl.BlockSpec((Q, 8), lambda t: (0, 0)),
            pl.BlockSpec((Q, 8), lambda t: (0, 0)),
        ],
        out_shape=[
            jax.ShapeDtypeStruct((T, Q, S, 128), jnp.float32),
            jax.ShapeDtypeStruct((Q, 8), jnp.int32),
            jax.ShapeDtypeStruct((Q, 8), jnp.int32),
        ],
        scratch_shapes=[
            pltpu.VMEM((S, Q), jnp.float32),
            pltpu.VMEM((8, Q), jnp.float32),
            pltpu.VMEM((8, Q), jnp.int32),
        ],
        compiler_params=pltpu.CompilerParams(
            dimension_semantics=("arbitrary",),
        ),
    )(queries, kp)


def _sc_gather(sim_flat, flat_idx):
    mesh = plsc.VectorSubcoreMesh(core_axis_name="c", subcore_axis_name="s")

    @functools.partial(
        pl.kernel,
        mesh=mesh,
        out_type=jax.ShapeDtypeStruct((Q * 8, 128), jnp.float32),
        scratch_types=[
            pltpu.VMEM((2, 128), jnp.int32),
            pltpu.VMEM((RPW, 128), jnp.float32),
            pltpu.SemaphoreType.DMA,
        ],
    )
    def k(sim_hbm, idx_hbm, out_hbm, idx_v, rows_v, sem):
        wid = lax.axis_index("s") * 2 + lax.axis_index("c")
        base = wid * RPW
        for g in range(2):
            pltpu.sync_copy(idx_hbm.at[pl.ds(base + g * 128, 128)], idx_v.at[g])
            pltpu.async_copy(
                sim_hbm.at[idx_v.at[g]],
                rows_v.at[pl.ds(g * 128, 128)],
                sem,
            ).wait()
        pltpu.sync_copy(rows_v, out_hbm.at[pl.ds(base, RPW)])

    return k(sim_flat, flat_idx)


def _stage_c(cand_ref, blk_ref, outv_ref, outi_ref):
    cv = cand_ref[...]
    blk = blk_ref[...]
    off = lax.broadcasted_iota(jnp.int32, (Q, 8, 128), 2)
    gidx = blk[:, :, None] * 128 + off
    mv, mi = [], []
    for _ in range(8):
        m = jnp.max(jnp.max(cv, axis=2), axis=1)
        eqi = jnp.where(cv == m[:, None, None], gidx, BIGI)
        pick = jnp.min(jnp.min(eqi, axis=2), axis=1)
        mv.append(m[:, None])
        mi.append(pick[:, None])
        cv = jnp.where(gidx == pick[:, None, None], NEG, cv)
    outv_ref[...] = jnp.concatenate(mv, axis=1)
    outi_ref[...] = jnp.concatenate(mi, axis=1)


def _stage_c_call(cand, blk):
    return pl.pallas_call(
        _stage_c,
        out_shape=[
            jax.ShapeDtypeStruct((Q, 8), jnp.float32),
            jax.ShapeDtypeStruct((Q, 8), jnp.int32),
        ],
    )(cand, blk)


def kernel(queries, keys, top_k):
    kp = jnp.pad(keys, ((0, KPAD - KTOT), (0, 0)))
    sim4, blk, flat = _stage_a_call(queries, kp)
    cand = _sc_gather(sim4.reshape(Q * NB, 128), flat.reshape(Q * 8))
    outv, outi = _stage_c_call(cand.reshape(Q, 8, 128), blk)
    return outv, outi + jnp.asarray(top_k - 8, jnp.int32)


# speedup vs baseline: 1.0329x; 1.0329x over previous
import functools

import jax
import jax.numpy as jnp
from jax import lax
from jax.experimental import pallas as pl
from jax.experimental.pallas import tpu as pltpu
from jax.experimental.pallas import tpu_sc as plsc

Q = 1024
D = 128
KTOT = 100000
W = 2048
S = W // 128
T = (KTOT + W - 1) // W
KPAD = T * W
NB = KPAD // 128
NEG = float("-inf")
BIGI = 2**30

NWORK = 32
RPW = (Q * 8) // NWORK


def _stage_a(q_ref, k_ref, sim_ref, blk_ref, flat_ref, bm_s):
    t = pl.program_id(0)
    q = q_ref[...]
    qn = q / jnp.maximum(jnp.sqrt(jnp.sum(q * q, axis=1, keepdims=True)), 1e-8)
    k = k_ref[...]
    kn = k / jnp.maximum(jnp.sqrt(jnp.sum(k * k, axis=1, keepdims=True)), 1e-8)
    sim = lax.dot_general(
        qn, kn, (((1,), (1,)), ((), ())),
        preferred_element_type=jnp.float32,
    )
    simt = lax.dot_general(
        kn, qn, (((1,), (1,)), ((), ())),
        preferred_element_type=jnp.float32,
    )

    @pl.when(t < T - 1)
    def _store_full():
        for j in range(S):
            sim_ref[0, :, j, :] = sim[:, j * 128:(j + 1) * 128]
        for j in range(S):
            bm_s[t, j, :] = jnp.max(simt[j * 128:(j + 1) * 128, :], axis=0)

    @pl.when(t == T - 1)
    def _store_masked():
        col = lax.broadcasted_iota(jnp.int32, (Q, W), 1)
        simm = jnp.where(col + t * W < KTOT, sim, NEG)
        for j in range(S):
            sim_ref[0, :, j, :] = simm[:, j * 128:(j + 1) * 128]
        rowi = lax.broadcasted_iota(jnp.int32, (W, Q), 0)
        simtm = jnp.where(rowi + t * W < KTOT, simt, NEG)
        for j in range(S):
            bm_s[t, j, :] = jnp.max(simtm[j * 128:(j + 1) * 128, :], axis=0)

    @pl.when(t == T - 1)
    def _select():
        bm = bm_s[...]
        bid = (lax.broadcasted_iota(jnp.int32, (T, S, Q), 0) * S
               + lax.broadcasted_iota(jnp.int32, (T, S, Q), 1))
        mv, mi = [], []
        for _ in range(8):
            m = jnp.max(jnp.max(bm, axis=0), axis=0)
            eqi = jnp.where(bm == m[None, None, :], bid, BIGI)
            pick = jnp.min(jnp.min(eqi, axis=0), axis=0)
            mv.append(m)
            mi.append(pick)
            bm = jnp.where(bid == pick[None, None, :], NEG, bm)
        blk = jnp.stack(mi, axis=1)
        blk_ref[...] = blk
        rows = lax.broadcasted_iota(jnp.int32, (Q, 8), 0)
        flat_ref[...] = (blk // S) * (Q * S) + rows * S + (blk % S)


def _stage_a_call(queries, kp):
    return pl.pallas_call(
        _stage_a,
        grid=(T,),
        in_specs=[
            pl.BlockSpec((Q, D), lambda t: (0, 0)),
            pl.BlockSpec((W, D), lambda t: (t, 0)),
        ],
        out_specs=[
            pl.BlockSpec((1, Q, S, 128), lambda t: (t, 0, 0, 0)),
            pl.BlockSpec((Q, 8), lambda t: (0, 0)),
            pl.BlockSpec((Q, 8), lambda t: (0, 0)),
        ],
        out_shape=[
            jax.ShapeDtypeStruct((T, Q, S, 128), jnp.float32),
            jax.ShapeDtypeStruct((Q, 8), jnp.int32),
            jax.ShapeDtypeStruct((Q, 8), jnp.int32),
        ],
        scratch_shapes=[
            pltpu.VMEM((T, S, Q), jnp.float32),
        ],
        compiler_params=pltpu.CompilerParams(
            dimension_semantics=("arbitrary",),
        ),
    )(queries, kp)


def _sc_gather(sim_flat, flat_idx):
    mesh = plsc.VectorSubcoreMesh(core_axis_name="c", subcore_axis_name="s")

    @functools.partial(
        pl.kernel,
        mesh=mesh,
        out_type=jax.ShapeDtypeStruct((Q * 8, 128), jnp.float32),
        scratch_types=[
            pltpu.VMEM((2, 128), jnp.int32),
            pltpu.VMEM((RPW, 128), jnp.float32),
            pltpu.SemaphoreType.DMA,
        ],
    )
    def k(sim_hbm, idx_hbm, out_hbm, idx_v, rows_v, sem):
        wid = lax.axis_index("s") * 2 + lax.axis_index("c")
        base = wid * RPW
        for g in range(2):
            pltpu.sync_copy(idx_hbm.at[pl.ds(base + g * 128, 128)], idx_v.at[g])
            pltpu.async_copy(
                sim_hbm.at[idx_v.at[g]],
                rows_v.at[pl.ds(g * 128, 128)],
                sem,
            ).wait()
        pltpu.sync_copy(rows_v, out_hbm.at[pl.ds(base, RPW)])

    return k(sim_flat, flat_idx)


def _stage_c(cand_ref, blk_ref, outv_ref, outi_ref):
    cv = cand_ref[...]
    blk = blk_ref[...]
    off = lax.broadcasted_iota(jnp.int32, (Q, 8, 128), 2)
    gidx = blk[:, :, None] * 128 + off
    mv, mi = [], []
    for _ in range(8):
        m = jnp.max(jnp.max(cv, axis=2), axis=1)
        eqi = jnp.where(cv == m[:, None, None], gidx, BIGI)
        pick = jnp.min(jnp.min(eqi, axis=2), axis=1)
        mv.append(m[:, None])
        mi.append(pick[:, None])
        cv = jnp.where(gidx == pick[:, None, None], NEG, cv)
    outv_ref[...] = jnp.concatenate(mv, axis=1)
    outi_ref[...] = jnp.concatenate(mi, axis=1)


def _stage_c_call(cand, blk):
    return pl.pallas_call(
        _stage_c,
        out_shape=[
            jax.ShapeDtypeStruct((Q, 8), jnp.float32),
            jax.ShapeDtypeStruct((Q, 8), jnp.int32),
        ],
    )(cand, blk)


def kernel(queries, keys, top_k):
    kp = jnp.pad(keys, ((0, KPAD - KTOT), (0, 0)))
    sim4, blk, flat = _stage_a_call(queries, kp)
    cand = _sc_gather(sim4.reshape(Q * NB, 128), flat.reshape(Q * 8))
    outv, outi = _stage_c_call(cand.reshape(Q, 8, 128), blk)
    return outv, outi + jnp.asarray(top_k - 8, jnp.int32)


# speedup vs baseline: 1.2184x; 1.1796x over previous
import functools

import jax
import jax.numpy as jnp
from jax import lax
from jax.experimental import pallas as pl
from jax.experimental.pallas import tpu as pltpu
from jax.experimental.pallas import tpu_sc as plsc

Q = 1024
D = 128
KTOT = 100000
W = 2048
S = W // 128
T = (KTOT + W - 1) // W
KPAD = T * W
NB = KPAD // 128
NEG = float("-inf")
BIGI = 2**30

NWORK = 32
RPW = (Q * 8) // NWORK


def _stage_a(q_ref, k_ref, sim_ref, blk_ref, flat_ref, bm_s):
    t = pl.program_id(0)
    q = q_ref[...]
    qn = q / jnp.maximum(jnp.sqrt(jnp.sum(q * q, axis=1, keepdims=True)), 1e-8)
    k = k_ref[...]
    kn = k / jnp.maximum(jnp.sqrt(jnp.sum(k * k, axis=1, keepdims=True)), 1e-8)
    sim = lax.dot_general(
        qn, kn, (((1,), (1,)), ((), ())),
        preferred_element_type=jnp.float32,
    )
    col = lax.broadcasted_iota(jnp.int32, (Q, W), 1)
    sim = jnp.where(col + t * W < KTOT, sim, NEG)

    for j in range(S):
        sim_ref[0, :, j, :] = sim[:, j * 128:(j + 1) * 128]

    simt = lax.dot_general(
        kn, qn, (((1,), (1,)), ((), ())),
        preferred_element_type=jnp.float32,
    )
    rowi = lax.broadcasted_iota(jnp.int32, (W, Q), 0)
    simt = jnp.where(rowi + t * W < KTOT, simt, NEG)
    for j in range(S):
        bm_s[t, j, :] = jnp.max(simt[j * 128:(j + 1) * 128, :], axis=0)

    @pl.when(t == T - 1)
    def _select():
        bm = bm_s[...]
        bid = (lax.broadcasted_iota(jnp.int32, (T, S, Q), 0) * S
               + lax.broadcasted_iota(jnp.int32, (T, S, Q), 1))
        mv, mi = [], []
        for _ in range(8):
            m = jnp.max(jnp.max(bm, axis=0), axis=0)
            eqi = jnp.where(bm == m[None, None, :], bid, BIGI)
            pick = jnp.min(jnp.min(eqi, axis=0), axis=0)
            mv.append(m)
            mi.append(pick)
            bm = jnp.where(bid == pick[None, None, :], NEG, bm)
        blk = jnp.stack(mi, axis=1)
        blk_ref[...] = blk
        rows = lax.broadcasted_iota(jnp.int32, (Q, 8), 0)
        flat_ref[...] = (blk // S) * (Q * S) + rows * S + (blk % S)


def _stage_a_call(queries, kp):
    return pl.pallas_call(
        _stage_a,
        grid=(T,),
        in_specs=[
            pl.BlockSpec((Q, D), lambda t: (0, 0)),
            pl.BlockSpec((W, D), lambda t: (t, 0)),
        ],
        out_specs=[
            pl.BlockSpec((1, Q, S, 128), lambda t: (t, 0, 0, 0)),
            pl.BlockSpec((Q, 8), lambda t: (0, 0)),
            pl.BlockSpec((Q, 8), lambda t: (0, 0)),
        ],
        out_shape=[
            jax.ShapeDtypeStruct((T, Q, S, 128), jnp.float32),
            jax.ShapeDtypeStruct((Q, 8), jnp.int32),
            jax.ShapeDtypeStruct((Q, 8), jnp.int32),
        ],
        scratch_shapes=[
            pltpu.VMEM((T, S, Q), jnp.float32),
        ],
        compiler_params=pltpu.CompilerParams(
            dimension_semantics=("arbitrary",),
        ),
    )(queries, kp)


def _sc_gather(sim_flat, flat_idx):
    mesh = plsc.VectorSubcoreMesh(core_axis_name="c", subcore_axis_name="s")

    @functools.partial(
        pl.kernel,
        mesh=mesh,
        out_type=jax.ShapeDtypeStruct((Q * 8, 128), jnp.float32),
        scratch_types=[
            pltpu.VMEM((2, 128), jnp.int32),
            pltpu.VMEM((RPW, 128), jnp.float32),
            pltpu.SemaphoreType.DMA,
        ],
    )
    def k(sim_hbm, idx_hbm, out_hbm, idx_v, rows_v, sem):
        wid = lax.axis_index("s") * 2 + lax.axis_index("c")
        base = wid * RPW
        for g in range(2):
            pltpu.sync_copy(idx_hbm.at[pl.ds(base + g * 128, 128)], idx_v.at[g])
            pltpu.async_copy(
                sim_hbm.at[idx_v.at[g]],
                rows_v.at[pl.ds(g * 128, 128)],
                sem,
            ).wait()
        pltpu.sync_copy(rows_v, out_hbm.at[pl.ds(base, RPW)])

    return k(sim_flat, flat_idx)


def _stage_c(cand_ref, blk_ref, outv_ref, outi_ref):
    cv = cand_ref[...]
    blk = blk_ref[...]
    off = lax.broadcasted_iota(jnp.int32, (Q, 8, 128), 2)
    gidx = blk[:, :, None] * 128 + off
    mv, mi = [], []
    for _ in range(8):
        m = jnp.max(jnp.max(cv, axis=2), axis=1)
        eqi = jnp.where(cv == m[:, None, None], gidx, BIGI)
        pick = jnp.min(jnp.min(eqi, axis=2), axis=1)
        mv.append(m[:, None])
        mi.append(pick[:, None])
        cv = jnp.where(gidx == pick[:, None, None], NEG, cv)
    outv_ref[...] = jnp.concatenate(mv, axis=1)
    outi_ref[...] = jnp.concatenate(mi, axis=1)


def _stage_c_call(cand, blk):
    return pl.pallas_call(
        _stage_c,
        out_shape=[
            jax.ShapeDtypeStruct((Q, 8), jnp.float32),
            jax.ShapeDtypeStruct((Q, 8), jnp.int32),
        ],
    )(cand, blk)


def kernel(queries, keys, top_k):
    kp = jnp.pad(keys, ((0, KPAD - KTOT), (0, 0)))
    sim4, blk, flat = _stage_a_call(queries, kp)
    cand = _sc_gather(sim4.reshape(Q * NB, 128), flat.reshape(Q * 8))
    outv, outi = _stage_c_call(cand.reshape(Q, 8, 128), blk)
    return outv, outi + jnp.asarray(top_k - 8, jnp.int32)
